# baseline (speedup 1.0000x reference)
import functools

import jax
import jax.numpy as jnp
from jax import lax
from jax.experimental import pallas as pl
from jax.experimental.pallas import tpu as pltpu
from jax.experimental.pallas import tpu_sc as plsc

ENT_TOTAL = 1000000
REL_TOTAL = 1000
HIDDEN = 16
BATCH = 16384
LMBDA = 0.1

NC = 2
NS = 16
NW = NC * NS
BPW = BATCH // NW
CHUNK = 128
NCHUNK = BPW // CHUNK
IDROWS = BATCH // CHUNK


def _sc_body(idh_hbm, idt_hbm, idr_hbm, emb_hbm, ere_hbm, eim_hbm,
             remb_hbm, rre_hbm, rim_hbm,
             c_hbm, reg_hbm,
             idh_v, idt_v, idr_v,
             g_erh, g_eih, g_eh, g_ert, g_eit, g_et, g_rre, g_rim, g_r,
             c_v, reg_v, sem):
  wid = lax.axis_index("s") * NC + lax.axis_index("c")
  base = wid * BPW
  row0 = wid * NCHUNK

  pltpu.sync_copy(idh_hbm.at[pl.ds(row0, NCHUNK)], idh_v)
  pltpu.sync_copy(idt_hbm.at[pl.ds(row0, NCHUNK)], idt_v)
  pltpu.sync_copy(idr_hbm.at[pl.ds(row0, NCHUNK)], idr_v)

  accs = [jnp.zeros((HIDDEN,), jnp.float32) for _ in range(9)]

  for j in range(NCHUNK):
    sl = pl.ds(j * CHUNK, CHUNK)
    cps = [
        pltpu.async_copy(ere_hbm.at[idh_v.at[j]], g_erh.at[sl], sem),
        pltpu.async_copy(eim_hbm.at[idh_v.at[j]], g_eih.at[sl], sem),
        pltpu.async_copy(emb_hbm.at[idh_v.at[j]], g_eh.at[sl], sem),
        pltpu.async_copy(ere_hbm.at[idt_v.at[j]], g_ert.at[sl], sem),
        pltpu.async_copy(eim_hbm.at[idt_v.at[j]], g_eit.at[sl], sem),
        pltpu.async_copy(emb_hbm.at[idt_v.at[j]], g_et.at[sl], sem),
        pltpu.async_copy(rre_hbm.at[idr_v.at[j]], g_rre.at[sl], sem),
        pltpu.async_copy(rim_hbm.at[idr_v.at[j]], g_rim.at[sl], sem),
        pltpu.async_copy(remb_hbm.at[idr_v.at[j]], g_r.at[sl], sem),
    ]
    for cp in cps:
      cp.wait()

    def body(i, accs, j=j):
      row = j * CHUNK + i
      erh = g_erh[row, :]
      eih = g_eih[row, :]
      eh = g_eh[row, :]
      ert = g_ert[row, :]
      eit = g_eit[row, :]
      et = g_et[row, :]
      rre = g_rre[row, :]
      rim = g_rim[row, :]
      r = g_r[row, :]
      cvec = (rre * (erh * ert + eih * eit)
              + rim * (erh * eit - eih * ert)
              + eh * et * r)
      c_v[row, :] = cvec
      vals = (erh, eih, eh, ert, eit, et, rre, rim, r)
      return tuple(a + v * v for a, v in zip(accs, vals))

    accs = lax.fori_loop(0, CHUNK, body, tuple(accs))

  for k in range(9):
    reg_v[k, :] = accs[k]
  pltpu.sync_copy(c_v, c_hbm.at[pl.ds(base, BPW)])
  pltpu.sync_copy(reg_v, reg_hbm.at[wid])


@jax.jit
def _sc_call(idh, idt, idr, emb, ere, eim, remb, rre, rim):
  mesh = plsc.VectorSubcoreMesh(core_axis_name="c", subcore_axis_name="s")
  f = pl.kernel(
      _sc_body,
      out_type=(
          jax.ShapeDtypeStruct((BATCH, HIDDEN), jnp.float32),
          jax.ShapeDtypeStruct((NW, 9, HIDDEN), jnp.float32),
      ),
      mesh=mesh,
      scratch_types=[
          pltpu.VMEM((NCHUNK, CHUNK), jnp.int32),
          pltpu.VMEM((NCHUNK, CHUNK), jnp.int32),
          pltpu.VMEM((NCHUNK, CHUNK), jnp.int32),
      ] + [pltpu.VMEM((BPW, HIDDEN), jnp.float32) for _ in range(9)] + [
          pltpu.VMEM((BPW, HIDDEN), jnp.float32),
          pltpu.VMEM((9, HIDDEN), jnp.float32),
          pltpu.SemaphoreType.DMA,
      ],
      compiler_params=pltpu.CompilerParams(use_tc_tiling_on_sc=False),
  )
  return f(idh, idt, idr, emb, ere, eim, remb, rre, rim)


def _tc_body(c_ref, y_ref, reg_ref, out_ref):
  res = jnp.sum(c_ref[...], axis=1, keepdims=True)
  x = -(y_ref[...] * res)
  sp = jnp.maximum(x, 0.0) + jnp.log(1.0 + jnp.exp(-jnp.abs(x)))
  loss = jnp.sum(sp) * (1.0 / BATCH)
  reg = reg_ref[...]
  scale = 1.0 / (BATCH * HIDDEN)
  m = [jnp.sum(reg[:, k * HIDDEN:(k + 1) * HIDDEN]) * scale for k in range(9)]
  regul = m[0] + m[1] * m[2] + m[3] + m[4] + m[5] + m[6] + m[7] + m[8]
  out_ref[...] = jnp.full((1, 1), loss + LMBDA * regul, jnp.float32)


@jax.jit
def _tc_call(c, y2, reg2):
  return pl.pallas_call(
      _tc_body,
      out_shape=jax.ShapeDtypeStruct((1, 1), jnp.float32),
  )(c, y2, reg2)


def kernel(id_h, id_t, id_r, y, ent_embeddings, ent_re, ent_im,
           rel_embeddings, rel_re, rel_im):
  idh = id_h.astype(jnp.int32).reshape(IDROWS, CHUNK)
  idt = id_t.astype(jnp.int32).reshape(IDROWS, CHUNK)
  idr = id_r.astype(jnp.int32).reshape(IDROWS, CHUNK)
  c, regp = _sc_call(idh, idt, idr, ent_embeddings, ent_re, ent_im,
                     rel_embeddings, rel_re, rel_im)
  out = _tc_call(c, y.reshape(BATCH, 1), regp.reshape(NW, 9 * HIDDEN))
  return out[0, 0]
